# Initial kernel scaffold; baseline (speedup 1.0000x reference)
import jax
import jax.numpy as jnp
from jax.experimental import pallas as pl


def kernel(x, edge_index, batch, edge_attr, W0, b0, enW1, enb1, enW2, enb2, root, conv_bias, gru_Wih, gru_Whh, gru_bih, gru_bhh, ls_Wih, ls_Whh, ls_bih, ls_bhh, W1, b1, W2, b2):
    raise NotImplementedError("write your pallas kernel here")


# speedup vs baseline: 2.3199x; 2.3199x over previous
import functools

import jax
import jax.numpy as jnp
from jax import lax
from jax.experimental import pallas as pl
from jax.experimental.pallas import tpu as pltpu
import jax.experimental.pallas.tpu_sc as plsc

_N, _E, _F, _H, _B = 5000, 30000, 128, 64, 256
_NC, _NS = 2, 16
_NW = _NC * _NS
_CW = 120
_CH = 8
_BPW = _CW * _CH
_EP = _BPW * _NW
_NP = 5120
_NPS = _NP // _NS
_MB = 512


def _prep_body(x_ref, w0_ref, b0_ref, ea_ref, w1_ref, b1_ref, s0_ref, eh_ref):
    s0_ref[...] = jax.nn.relu(
        jnp.dot(x_ref[...], w0_ref[...], preferred_element_type=jnp.float32)
        + b0_ref[...])
    eh = jax.nn.relu(
        jnp.dot(ea_ref[...], w1_ref[...], preferred_element_type=jnp.float32)
        + b1_ref[...])
    eh_ref[...] = eh.astype(jnp.bfloat16)


def _prep(x, w0, b0, ea_p, enw1, enb1):
    return pl.pallas_call(
        _prep_body,
        out_shape=(jax.ShapeDtypeStruct((_N, _H), jnp.float32),
                   jax.ShapeDtypeStruct((_EP, 128), jnp.bfloat16)),
    )(x, w0, b0, ea_p, enw1, enb1)


def _gather_body(nodes_hbm, idx_hbm, g_hbm, idx_v, rows_v, sem):
    c = lax.axis_index("c")
    s = lax.axis_index("s")
    wid = s * _NC + c
    pltpu.sync_copy(idx_hbm.at[wid], idx_v)
    cps = [
        pltpu.async_copy(nodes_hbm.at[idx_v.at[j]],
                         rows_v.at[pl.ds(j * _CW, _CW)], sem)
        for j in range(_CH)
    ]
    for cp in cps:
        cp.wait()
    pltpu.sync_copy(rows_v, g_hbm.at[pl.ds(wid * _BPW, _BPW)])


def _sc_gather(nodes, src_idx):
    mesh = plsc.VectorSubcoreMesh(core_axis_name="c", subcore_axis_name="s")
    return pl.kernel(
        _gather_body,
        out_type=jax.ShapeDtypeStruct((_EP, _H), jnp.float32),
        mesh=mesh,
        compiler_params=pltpu.CompilerParams(use_tc_tiling_on_sc=False),
        scratch_types=[
            pltpu.VMEM((_CH, _CW), jnp.int32),
            pltpu.VMEM((_BPW, _H), jnp.float32),
            pltpu.SemaphoreType.DMA,
        ],
    )(nodes, src_idx)


def _msg_body(eh_ref, g_ref, w_ref, b_ref, o_ref):
    ehb = eh_ref[...]
    g32 = g_ref[...]
    gb = g32.astype(jnp.bfloat16)
    p = (gb[:, :, None] * ehb[:, None, :]).reshape(_MB, _H * 128)
    acc = jnp.dot(p, w_ref[...], preferred_element_type=jnp.float32)
    acc = acc + jnp.dot(g32, b_ref[...], preferred_element_type=jnp.float32)
    o_ref[...] = acc


def _tc_msg(ehb, g, w2rb, b0m):
    grid = _EP // _MB
    return pl.pallas_call(
        _msg_body,
        grid=(grid,),
        in_specs=[
            pl.BlockSpec((_MB, 128), lambda i: (i, 0)),
            pl.BlockSpec((_MB, _H), lambda i: (i, 0)),
            pl.BlockSpec((_H * 128, _H), lambda i: (0, 0)),
            pl.BlockSpec((_H, _H), lambda i: (0, 0)),
        ],
        out_specs=pl.BlockSpec((_MB, _H), lambda i: (i, 0)),
        out_shape=jax.ShapeDtypeStruct((_EP, _H), jnp.float32),
    )(ehb, g, w2rb, b0m)


def _scatter_body(msg_hbm, idx_hbm, z64_hbm, z16_hbm, ones_hbm,
                  agg_hbm, deg_hbm, idx_v, msg_v, ones_v, t16_v,
                  agg_sh, deg_sh):
    c = lax.axis_index("c")
    s = lax.axis_index("s")
    wid = s * _NC + c
    row0 = s * _NPS
    pltpu.sync_copy(z64_hbm.at[pl.ds(row0, _NPS)], msg_v.at[pl.ds(0, _NPS)])
    pltpu.sync_copy(msg_v.at[pl.ds(0, _NPS)], agg_sh.at[pl.ds(row0, _NPS)])
    pltpu.sync_copy(z16_hbm.at[pl.ds(row0, _NPS)], t16_v)
    pltpu.sync_copy(t16_v, deg_sh.at[pl.ds(row0, _NPS)])
    pltpu.sync_copy(ones_hbm, ones_v)
    plsc.subcore_barrier()
    pltpu.sync_copy(idx_hbm.at[wid], idx_v)
    pltpu.sync_copy(msg_hbm.at[pl.ds(wid * _BPW, _BPW)], msg_v)
    for j in range(_CH):
        pltpu.sync_copy(msg_v.at[pl.ds(j * _CW, _CW)],
                        agg_sh.at[idx_v.at[j]], add=True)
        pltpu.sync_copy(ones_v, deg_sh.at[idx_v.at[j]], add=True)
    plsc.subcore_barrier()
    pltpu.sync_copy(agg_sh.at[pl.ds(row0, _NPS)], msg_v.at[pl.ds(0, _NPS)])
    pltpu.sync_copy(msg_v.at[pl.ds(0, _NPS)],
                    agg_hbm.at[pl.ds(c * _NP + row0, _NPS)])
    pltpu.sync_copy(deg_sh.at[pl.ds(row0, _NPS)], t16_v)
    pltpu.sync_copy(t16_v, deg_hbm.at[pl.ds(c * _NP + row0, _NPS)])


def _sc_scatter(msg, dst_idx, z64, z16, ones):
    mesh = plsc.VectorSubcoreMesh(core_axis_name="c", subcore_axis_name="s")
    return pl.kernel(
        _scatter_body,
        out_type=(jax.ShapeDtypeStruct((_NC * _NP, _H), jnp.float32),
                  jax.ShapeDtypeStruct((_NC * _NP, 16), jnp.float32)),
        mesh=mesh,
        compiler_params=pltpu.CompilerParams(use_tc_tiling_on_sc=False),
        scratch_types=[
            pltpu.VMEM((_CH, _CW), jnp.int32),
            pltpu.VMEM((_BPW, _H), jnp.float32),
            pltpu.VMEM((_CW, 16), jnp.float32),
            pltpu.VMEM((_NPS, 16), jnp.float32),
            pltpu.VMEM_SHARED((_NP, _H), jnp.float32),
            pltpu.VMEM_SHARED((_NP, 16), jnp.float32),
        ],
    )(msg, dst_idx, z64, z16, ones)


def _gru_body(agg_ref, deg_ref, s_ref, root_ref, cb_ref, wih_ref, bih_ref,
              whh_ref, bhh_ref, o_ref):
    deg = jnp.clip(deg_ref[0:_N, 0:1] + deg_ref[_NP:_NP + _N, 0:1], 1.0, None)
    agg = (agg_ref[0:_N, :] + agg_ref[_NP:_NP + _N, :]) / deg
    s = s_ref[...]
    m = jax.nn.relu(
        agg + jnp.dot(s, root_ref[...], preferred_element_type=jnp.float32)
        + cb_ref[...])
    gi = jnp.dot(m, wih_ref[...], preferred_element_type=jnp.float32) + bih_ref[...]
    gh = jnp.dot(s, whh_ref[...], preferred_element_type=jnp.float32) + bhh_ref[...]
    r = jax.nn.sigmoid(gi[:, 0:_H] + gh[:, 0:_H])
    z = jax.nn.sigmoid(gi[:, _H:2 * _H] + gh[:, _H:2 * _H])
    n = jnp.tanh(gi[:, 2 * _H:3 * _H] + r * gh[:, 2 * _H:3 * _H])
    o_ref[...] = (1.0 - z) * n + z * s


def _tc_gru(aggp, degp, s, root, cb, wihT, bih, whhT, bhh):
    return pl.pallas_call(
        _gru_body,
        out_shape=jax.ShapeDtypeStruct((_N, _H), jnp.float32),
    )(aggp, degp, s, root, cb, wihT, bih, whhT, bhh)


def _s2s_body(s_ref, b_ref, wih_ref, bih_ref, whh_ref, bhh_ref,
              w1_ref, b1_ref, w2_ref, b2_ref, o_ref):
    s = s_ref[...]
    bids = b_ref[...]
    iota = lax.broadcasted_iota(jnp.int32, (_N, _B), 1)
    msk = bids == iota
    mf = msk.astype(jnp.float32)
    q = jnp.zeros((_B, 2 * _H), jnp.float32)
    hh = jnp.zeros((_B, _H), jnp.float32)
    cc = jnp.zeros((_B, _H), jnp.float32)
    for _ in range(3):
        gates = (jnp.dot(q, wih_ref[...], preferred_element_type=jnp.float32)
                 + bih_ref[...]
                 + jnp.dot(hh, whh_ref[...], preferred_element_type=jnp.float32)
                 + bhh_ref[...])
        i_ = jax.nn.sigmoid(gates[:, 0:_H])
        f_ = jax.nn.sigmoid(gates[:, _H:2 * _H])
        g_ = jnp.tanh(gates[:, 2 * _H:3 * _H])
        o_ = jax.nn.sigmoid(gates[:, 3 * _H:4 * _H])
        cc = f_ * cc + i_ * g_
        hh = o_ * jnp.tanh(cc)
        hb = jnp.dot(mf, hh, preferred_element_type=jnp.float32)
        e = jnp.sum(s * hb, axis=1, keepdims=True)
        em = jnp.max(jnp.where(msk, e, -1e38), axis=0, keepdims=True)
        em = jnp.where(em < -1e37, 0.0, em)
        a = jnp.exp(e - jnp.sum(mf * em, axis=1, keepdims=True))
        asum = jnp.sum(mf * a, axis=0, keepdims=True)
        an = a / (jnp.sum(mf * asum, axis=1, keepdims=True) + 1e-16)
        r_ = lax.dot_general(mf * an, s, (((0,), (0,)), ((), ())),
                             preferred_element_type=jnp.float32)
        q = jnp.concatenate([hh, r_], axis=1)
    z1 = jax.nn.relu(
        jnp.dot(q, w1_ref[...], preferred_element_type=jnp.float32) + b1_ref[...])
    o_ref[...] = jnp.dot(z1, w2_ref[...], preferred_element_type=jnp.float32) + b2_ref[...]


def _tc_s2s(s, batch2d, lsWihT, lsbih, lsWhhT, lsbhh, w1, b1, w2, b2):
    return pl.pallas_call(
        _s2s_body,
        out_shape=jax.ShapeDtypeStruct((_B, 1), jnp.float32),
    )(s, batch2d, lsWihT, lsbih, lsWhhT, lsbhh, w1, b1, w2, b2)


def kernel(x, edge_index, batch, edge_attr, W0, b0, enW1, enb1, enW2, enb2,
           root, conv_bias, gru_Wih, gru_Whh, gru_bih, gru_bhh,
           ls_Wih, ls_Whh, ls_bih, ls_bhh, W1, b1, W2, b2):
    src = edge_index[0].astype(jnp.int32)
    dst = edge_index[1].astype(jnp.int32)
    src_p = jnp.concatenate([src, jnp.zeros((_EP - _E,), jnp.int32)])
    dst_p = jnp.concatenate([dst, jnp.full((_EP - _E,), _N, jnp.int32)])
    src_w = src_p.reshape(_NW, _CH, _CW)
    dst_w = dst_p.reshape(_NW, _CH, _CW)
    ea_p = jnp.pad(edge_attr, ((0, _EP - _E), (0, 0)))

    w2rb = (enW2.reshape(128, _H, _H).transpose(1, 0, 2)
            .reshape(_H * 128, _H).astype(jnp.bfloat16))
    b0m = enb2.reshape(_H, _H)
    z64 = jnp.zeros((_NP, _H), jnp.float32)
    z16 = jnp.zeros((_NP, 16), jnp.float32)
    ones = jnp.ones((_CW, 16), jnp.float32)

    s, ehb = _prep(x, W0, b0, ea_p, enW1, enb1)

    wihT = gru_Wih.T
    whhT = gru_Whh.T
    for _ in range(3):
        g = _sc_gather(s, src_w)
        msg = _tc_msg(ehb, g, w2rb, b0m)
        aggp, degp = _sc_scatter(msg, dst_w, z64, z16, ones)
        s = _tc_gru(aggp, degp, s, root, conv_bias, wihT, gru_bih,
                    whhT, gru_bhh)

    out = _tc_s2s(s, batch.astype(jnp.int32).reshape(_N, 1),
                  ls_Wih.T, ls_bih, ls_Whh.T, ls_bhh, W1, b1, W2, b2)
    return out.reshape(-1)


# speedup vs baseline: 3.3721x; 1.4536x over previous
import functools

import jax
import jax.numpy as jnp
from jax import lax
from jax.experimental import pallas as pl
from jax.experimental.pallas import tpu as pltpu
import jax.experimental.pallas.tpu_sc as plsc

_N, _E, _F, _H, _B = 5000, 30000, 128, 64, 256
_NC, _NS = 2, 16
_NW = _NC * _NS
_CW = 120
_CH = 8
_BPW = _CW * _CH
_EP = _BPW * _NW
_NP = 5120
_NPS = _NP // _NS
_MB = 512


def _prep_body(x_ref, w0_ref, b0_ref, ea_ref, w1_ref, b1_ref, s0_ref, eh_ref):
    s0_ref[...] = jax.nn.relu(
        jnp.dot(x_ref[...], w0_ref[...], preferred_element_type=jnp.float32)
        + b0_ref[...])
    ehT = jax.nn.relu(
        lax.dot_general(w1_ref[...], ea_ref[...], (((0,), (1,)), ((), ())),
                        preferred_element_type=jnp.float32)
        + b1_ref[...][:, None])
    eh_ref[...] = ehT.astype(jnp.bfloat16)


def _prep(x, w0, b0, ea_p, enw1, enb1):
    return pl.pallas_call(
        _prep_body,
        out_shape=(jax.ShapeDtypeStruct((_N, _H), jnp.float32),
                   jax.ShapeDtypeStruct((128, _EP), jnp.bfloat16)),
    )(x, w0, b0, ea_p, enw1, enb1)


def _gather_body(nodes_hbm, idx_hbm, g_hbm, idx_v, rows_v, sem):
    c = lax.axis_index("c")
    s = lax.axis_index("s")
    wid = s * _NC + c
    pltpu.sync_copy(idx_hbm.at[wid], idx_v)
    cps = [
        pltpu.async_copy(nodes_hbm.at[idx_v.at[j]],
                         rows_v.at[pl.ds(j * _CW, _CW)], sem)
        for j in range(_CH)
    ]
    for cp in cps:
        cp.wait()
    pltpu.sync_copy(rows_v, g_hbm.at[pl.ds(wid * _BPW, _BPW)])


def _sc_gather(nodes, src_idx):
    mesh = plsc.VectorSubcoreMesh(core_axis_name="c", subcore_axis_name="s")
    return pl.kernel(
        _gather_body,
        out_type=jax.ShapeDtypeStruct((_EP, _H), jnp.float32),
        mesh=mesh,
        compiler_params=pltpu.CompilerParams(use_tc_tiling_on_sc=False),
        scratch_types=[
            pltpu.VMEM((_CH, _CW), jnp.int32),
            pltpu.VMEM((_BPW, _H), jnp.float32),
            pltpu.SemaphoreType.DMA,
        ],
    )(nodes, src_idx)


def _msg_body(eh_ref, g_ref, w_ref, b_ref, o_ref):
    g32 = g_ref[...]
    gT = g32.T.astype(jnp.bfloat16)
    ehT = eh_ref[...]
    p2 = (gT[:, None, :] * ehT[None, :, :]).reshape(_H * 128, _MB)
    msgT = jnp.dot(w_ref[...], p2, preferred_element_type=jnp.float32)
    acc = msgT.T + jnp.dot(g32, b_ref[...], preferred_element_type=jnp.float32)
    o_ref[...] = acc


def _tc_msg(ehb, g, w2rb, b0m):
    grid = _EP // _MB
    return pl.pallas_call(
        _msg_body,
        grid=(grid,),
        in_specs=[
            pl.BlockSpec((128, _MB), lambda i: (0, i)),
            pl.BlockSpec((_MB, _H), lambda i: (i, 0)),
            pl.BlockSpec((_H, _H * 128), lambda i: (0, 0)),
            pl.BlockSpec((_H, _H), lambda i: (0, 0)),
        ],
        out_specs=pl.BlockSpec((_MB, _H), lambda i: (i, 0)),
        out_shape=jax.ShapeDtypeStruct((_EP, _H), jnp.float32),
    )(ehb, g, w2rb, b0m)


def _scatter_body(msg_hbm, idx_hbm, z64_hbm, z16_hbm, ones_hbm,
                  agg_hbm, deg_hbm, idx_v, msg_v, ones_v, t16_v,
                  agg_sh, deg_sh):
    c = lax.axis_index("c")
    s = lax.axis_index("s")
    wid = s * _NC + c
    row0 = s * _NPS
    pltpu.sync_copy(z64_hbm.at[pl.ds(row0, _NPS)], msg_v.at[pl.ds(0, _NPS)])
    pltpu.sync_copy(msg_v.at[pl.ds(0, _NPS)], agg_sh.at[pl.ds(row0, _NPS)])
    pltpu.sync_copy(z16_hbm.at[pl.ds(row0, _NPS)], t16_v)
    pltpu.sync_copy(t16_v, deg_sh.at[pl.ds(row0, _NPS)])
    pltpu.sync_copy(ones_hbm, ones_v)
    plsc.subcore_barrier()
    pltpu.sync_copy(idx_hbm.at[wid], idx_v)
    pltpu.sync_copy(msg_hbm.at[pl.ds(wid * _BPW, _BPW)], msg_v)
    for j in range(_CH):
        pltpu.sync_copy(msg_v.at[pl.ds(j * _CW, _CW)],
                        agg_sh.at[idx_v.at[j]], add=True)
        pltpu.sync_copy(ones_v, deg_sh.at[idx_v.at[j]], add=True)
    plsc.subcore_barrier()
    pltpu.sync_copy(agg_sh.at[pl.ds(row0, _NPS)], msg_v.at[pl.ds(0, _NPS)])
    pltpu.sync_copy(msg_v.at[pl.ds(0, _NPS)],
                    agg_hbm.at[pl.ds(c * _NP + row0, _NPS)])
    pltpu.sync_copy(deg_sh.at[pl.ds(row0, _NPS)], t16_v)
    pltpu.sync_copy(t16_v, deg_hbm.at[pl.ds(c * _NP + row0, _NPS)])


def _sc_scatter(msg, dst_idx, z64, z16, ones):
    mesh = plsc.VectorSubcoreMesh(core_axis_name="c", subcore_axis_name="s")
    return pl.kernel(
        _scatter_body,
        out_type=(jax.ShapeDtypeStruct((_NC * _NP, _H), jnp.float32),
                  jax.ShapeDtypeStruct((_NC * _NP, 16), jnp.float32)),
        mesh=mesh,
        compiler_params=pltpu.CompilerParams(use_tc_tiling_on_sc=False),
        scratch_types=[
            pltpu.VMEM((_CH, _CW), jnp.int32),
            pltpu.VMEM((_BPW, _H), jnp.float32),
            pltpu.VMEM((_CW, 16), jnp.float32),
            pltpu.VMEM((_NPS, 16), jnp.float32),
            pltpu.VMEM_SHARED((_NP, _H), jnp.float32),
            pltpu.VMEM_SHARED((_NP, 16), jnp.float32),
        ],
    )(msg, dst_idx, z64, z16, ones)


def _gru_body(agg_ref, deg_ref, s_ref, root_ref, cb_ref, wih_ref, bih_ref,
              whh_ref, bhh_ref, o_ref):
    deg = jnp.clip(deg_ref[0:_N, 0:1] + deg_ref[_NP:_NP + _N, 0:1], 1.0, None)
    agg = (agg_ref[0:_N, :] + agg_ref[_NP:_NP + _N, :]) / deg
    s = s_ref[...]
    m = jax.nn.relu(
        agg + jnp.dot(s, root_ref[...], preferred_element_type=jnp.float32)
        + cb_ref[...])
    gi = jnp.dot(m, wih_ref[...], preferred_element_type=jnp.float32) + bih_ref[...]
    gh = jnp.dot(s, whh_ref[...], preferred_element_type=jnp.float32) + bhh_ref[...]
    r = jax.nn.sigmoid(gi[:, 0:_H] + gh[:, 0:_H])
    z = jax.nn.sigmoid(gi[:, _H:2 * _H] + gh[:, _H:2 * _H])
    n = jnp.tanh(gi[:, 2 * _H:3 * _H] + r * gh[:, 2 * _H:3 * _H])
    o_ref[...] = (1.0 - z) * n + z * s


def _tc_gru(aggp, degp, s, root, cb, wihT, bih, whhT, bhh):
    return pl.pallas_call(
        _gru_body,
        out_shape=jax.ShapeDtypeStruct((_N, _H), jnp.float32),
    )(aggp, degp, s, root, cb, wihT, bih, whhT, bhh)


def _s2s_body(s_ref, b_ref, wih_ref, bih_ref, whh_ref, bhh_ref,
              w1_ref, b1_ref, w2_ref, b2_ref, o_ref):
    s = s_ref[...]
    bids = b_ref[...]
    iota = lax.broadcasted_iota(jnp.int32, (_N, _B), 1)
    msk = bids == iota
    mf = msk.astype(jnp.float32)
    q = jnp.zeros((_B, 2 * _H), jnp.float32)
    hh = jnp.zeros((_B, _H), jnp.float32)
    cc = jnp.zeros((_B, _H), jnp.float32)
    for _ in range(3):
        gates = (jnp.dot(q, wih_ref[...], preferred_element_type=jnp.float32)
                 + bih_ref[...]
                 + jnp.dot(hh, whh_ref[...], preferred_element_type=jnp.float32)
                 + bhh_ref[...])
        i_ = jax.nn.sigmoid(gates[:, 0:_H])
        f_ = jax.nn.sigmoid(gates[:, _H:2 * _H])
        g_ = jnp.tanh(gates[:, 2 * _H:3 * _H])
        o_ = jax.nn.sigmoid(gates[:, 3 * _H:4 * _H])
        cc = f_ * cc + i_ * g_
        hh = o_ * jnp.tanh(cc)
        hb = jnp.dot(mf, hh, preferred_element_type=jnp.float32)
        e = jnp.sum(s * hb, axis=1, keepdims=True)
        em = jnp.max(jnp.where(msk, e, -1e38), axis=0, keepdims=True)
        em = jnp.where(em < -1e37, 0.0, em)
        a = jnp.exp(e - jnp.sum(mf * em, axis=1, keepdims=True))
        asum = jnp.sum(mf * a, axis=0, keepdims=True)
        an = a / (jnp.sum(mf * asum, axis=1, keepdims=True) + 1e-16)
        r_ = lax.dot_general(mf * an, s, (((0,), (0,)), ((), ())),
                             preferred_element_type=jnp.float32)
        q = jnp.concatenate([hh, r_], axis=1)
    z1 = jax.nn.relu(
        jnp.dot(q, w1_ref[...], preferred_element_type=jnp.float32) + b1_ref[...])
    o_ref[...] = jnp.dot(z1, w2_ref[...], preferred_element_type=jnp.float32) + b2_ref[...]


def _tc_s2s(s, batch2d, lsWihT, lsbih, lsWhhT, lsbhh, w1, b1, w2, b2):
    return pl.pallas_call(
        _s2s_body,
        out_shape=jax.ShapeDtypeStruct((_B, 1), jnp.float32),
    )(s, batch2d, lsWihT, lsbih, lsWhhT, lsbhh, w1, b1, w2, b2)


def kernel(x, edge_index, batch, edge_attr, W0, b0, enW1, enb1, enW2, enb2,
           root, conv_bias, gru_Wih, gru_Whh, gru_bih, gru_bhh,
           ls_Wih, ls_Whh, ls_bih, ls_bhh, W1, b1, W2, b2):
    src = edge_index[0].astype(jnp.int32)
    dst = edge_index[1].astype(jnp.int32)
    src_p = jnp.concatenate([src, jnp.zeros((_EP - _E,), jnp.int32)])
    dst_p = jnp.concatenate([dst, jnp.full((_EP - _E,), _N, jnp.int32)])
    src_w = src_p.reshape(_NW, _CH, _CW)
    dst_w = dst_p.reshape(_NW, _CH, _CW)
    ea_p = jnp.pad(edge_attr, ((0, _EP - _E), (0, 0)))

    w2rb = (enW2.reshape(128, _H, _H).transpose(2, 1, 0)
            .reshape(_H, _H * 128).astype(jnp.bfloat16))
    b0m = enb2.reshape(_H, _H)
    z64 = jnp.zeros((_NP, _H), jnp.float32)
    z16 = jnp.zeros((_NP, 16), jnp.float32)
    ones = jnp.ones((_CW, 16), jnp.float32)

    s, ehb = _prep(x, W0, b0, ea_p, enW1, enb1)

    wihT = gru_Wih.T
    whhT = gru_Whh.T
    for _ in range(3):
        g = _sc_gather(s, src_w)
        msg = _tc_msg(ehb, g, w2rb, b0m)
        aggp, degp = _sc_scatter(msg, dst_w, z64, z16, ones)
        s = _tc_gru(aggp, degp, s, root, conv_bias, wihT, gru_bih,
                    whhT, gru_bhh)

    out = _tc_s2s(s, batch.astype(jnp.int32).reshape(_N, 1),
                  ls_Wih.T, ls_bih, ls_Whh.T, ls_bhh, W1, b1, W2, b2)
    return out.reshape(-1)


# speedup vs baseline: 3.6760x; 1.0901x over previous
import functools

import jax
import jax.numpy as jnp
from jax import lax
from jax.experimental import pallas as pl
from jax.experimental.pallas import tpu as pltpu
import jax.experimental.pallas.tpu_sc as plsc

_N, _E, _F, _H, _B = 5000, 30000, 128, 64, 256
_NC, _NS = 2, 16
_NW = _NC * _NS
_CW = 120
_CH = 8
_BPW = _CW * _CH
_EP = _BPW * _NW
_NP = 5120
_NPS = _NP // _NS
_MB = 512


def _prep_body(x_ref, w0_ref, b0_ref, ea_ref, w1_ref, b1_ref, s0_ref, eh_ref):
    s0_ref[0:_N, :] = jax.nn.relu(
        jnp.dot(x_ref[...], w0_ref[...], preferred_element_type=jnp.float32)
        + b0_ref[...])
    s0_ref[_N:_NP, :] = jnp.zeros((_NP - _N, _H), jnp.float32)
    ehT = jax.nn.relu(
        lax.dot_general(w1_ref[...], ea_ref[...], (((0,), (1,)), ((), ())),
                        preferred_element_type=jnp.float32)
        + b1_ref[...][:, None])
    eh_ref[...] = ehT.astype(jnp.bfloat16)


def _prep(x, w0, b0, ea_p, enw1, enb1):
    return pl.pallas_call(
        _prep_body,
        out_shape=(jax.ShapeDtypeStruct((_NP, _H), jnp.float32),
                   jax.ShapeDtypeStruct((128, _EP), jnp.bfloat16)),
    )(x, w0, b0, ea_p, enw1, enb1)


def _gather_body(nodes_hbm, idx_hbm, g_hbm, idx_v, rows_v, stage_v, tab_sh, sem):
    c = lax.axis_index("c")
    s = lax.axis_index("s")
    wid = s * _NC + c
    r0 = s * _NPS
    pltpu.sync_copy(nodes_hbm.at[pl.ds(r0, _NPS)], stage_v)
    pltpu.sync_copy(stage_v, tab_sh.at[pl.ds(r0, _NPS)])
    plsc.subcore_barrier()
    pltpu.sync_copy(idx_hbm.at[wid], idx_v)
    cps = [
        pltpu.async_copy(tab_sh.at[idx_v.at[j]],
                         rows_v.at[pl.ds(j * _CW, _CW)], sem)
        for j in range(_CH)
    ]
    for cp in cps:
        cp.wait()
    pltpu.sync_copy(rows_v, g_hbm.at[pl.ds(wid * _BPW, _BPW)])


def _sc_gather(nodes, src_idx):
    mesh = plsc.VectorSubcoreMesh(core_axis_name="c", subcore_axis_name="s")
    return pl.kernel(
        _gather_body,
        out_type=jax.ShapeDtypeStruct((_EP, _H), jnp.float32),
        mesh=mesh,
        compiler_params=pltpu.CompilerParams(use_tc_tiling_on_sc=False),
        scratch_types=[
            pltpu.VMEM((_CH, _CW), jnp.int32),
            pltpu.VMEM((_BPW, _H), jnp.float32),
            pltpu.VMEM((_NPS, _H), jnp.float32),
            pltpu.VMEM_SHARED((_NP, _H), jnp.float32),
            pltpu.SemaphoreType.DMA,
        ],
    )(nodes, src_idx)


def _msg_body(eh_ref, g_ref, w_ref, b_ref, o_ref):
    g32 = g_ref[...]
    gT = g32.T.astype(jnp.bfloat16)
    ehT = eh_ref[...]
    p2 = (gT[:, None, :] * ehT[None, :, :]).reshape(_H * 128, _MB)
    msgT = jnp.dot(w_ref[...], p2, preferred_element_type=jnp.float32)
    acc = msgT.T + jnp.dot(g32, b_ref[...], preferred_element_type=jnp.float32)
    o_ref[...] = acc


def _tc_msg(ehb, g, w2rb, b0m):
    grid = _EP // _MB
    return pl.pallas_call(
        _msg_body,
        grid=(grid,),
        in_specs=[
            pl.BlockSpec((128, _MB), lambda i: (0, i)),
            pl.BlockSpec((_MB, _H), lambda i: (i, 0)),
            pl.BlockSpec((_H, _H * 128), lambda i: (0, 0)),
            pl.BlockSpec((_H, _H), lambda i: (0, 0)),
        ],
        out_specs=pl.BlockSpec((_MB, _H), lambda i: (i, 0)),
        out_shape=jax.ShapeDtypeStruct((_EP, _H), jnp.float32),
    )(ehb, g, w2rb, b0m)


def _scatter_body(msg_hbm, idx_hbm, z64_hbm, z16_hbm, ones_hbm,
                  agg_hbm, deg_hbm, idx_v, msg_v, ones_v, t16_v,
                  agg_sh, deg_sh):
    c = lax.axis_index("c")
    s = lax.axis_index("s")
    wid = s * _NC + c
    row0 = s * _NPS
    pltpu.sync_copy(z64_hbm.at[pl.ds(row0, _NPS)], msg_v.at[pl.ds(0, _NPS)])
    pltpu.sync_copy(msg_v.at[pl.ds(0, _NPS)], agg_sh.at[pl.ds(row0, _NPS)])
    pltpu.sync_copy(z16_hbm.at[pl.ds(row0, _NPS)], t16_v)
    pltpu.sync_copy(t16_v, deg_sh.at[pl.ds(row0, _NPS)])
    pltpu.sync_copy(ones_hbm, ones_v)
    plsc.subcore_barrier()
    pltpu.sync_copy(idx_hbm.at[wid], idx_v)
    pltpu.sync_copy(msg_hbm.at[pl.ds(wid * _BPW, _BPW)], msg_v)
    for j in range(_CH):
        pltpu.sync_copy(msg_v.at[pl.ds(j * _CW, _CW)],
                        agg_sh.at[idx_v.at[j]], add=True)
        pltpu.sync_copy(ones_v, deg_sh.at[idx_v.at[j]], add=True)
    plsc.subcore_barrier()
    pltpu.sync_copy(agg_sh.at[pl.ds(row0, _NPS)], msg_v.at[pl.ds(0, _NPS)])
    pltpu.sync_copy(msg_v.at[pl.ds(0, _NPS)],
                    agg_hbm.at[pl.ds(c * _NP + row0, _NPS)])
    pltpu.sync_copy(deg_sh.at[pl.ds(row0, _NPS)], t16_v)
    pltpu.sync_copy(t16_v, deg_hbm.at[pl.ds(c * _NP + row0, _NPS)])


def _sc_scatter(msg, dst_idx, z64, z16, ones):
    mesh = plsc.VectorSubcoreMesh(core_axis_name="c", subcore_axis_name="s")
    return pl.kernel(
        _scatter_body,
        out_type=(jax.ShapeDtypeStruct((_NC * _NP, _H), jnp.float32),
                  jax.ShapeDtypeStruct((_NC * _NP, 16), jnp.float32)),
        mesh=mesh,
        compiler_params=pltpu.CompilerParams(use_tc_tiling_on_sc=False),
        scratch_types=[
            pltpu.VMEM((_CH, _CW), jnp.int32),
            pltpu.VMEM((_BPW, _H), jnp.float32),
            pltpu.VMEM((_CW, 16), jnp.float32),
            pltpu.VMEM((_NPS, 16), jnp.float32),
            pltpu.VMEM_SHARED((_NP, _H), jnp.float32),
            pltpu.VMEM_SHARED((_NP, 16), jnp.float32),
        ],
    )(msg, dst_idx, z64, z16, ones)


def _gru_body(agg_ref, deg_ref, s_ref, root_ref, cb_ref, wih_ref, bih_ref,
              whh_ref, bhh_ref, o_ref):
    deg = jnp.clip(deg_ref[0:_N, 0:1] + deg_ref[_NP:_NP + _N, 0:1], 1.0, None)
    agg = (agg_ref[0:_N, :] + agg_ref[_NP:_NP + _N, :]) / deg
    s = s_ref[0:_N, :]
    m = jax.nn.relu(
        agg + jnp.dot(s, root_ref[...], preferred_element_type=jnp.float32)
        + cb_ref[...])
    gi = jnp.dot(m, wih_ref[...], preferred_element_type=jnp.float32) + bih_ref[...]
    gh = jnp.dot(s, whh_ref[...], preferred_element_type=jnp.float32) + bhh_ref[...]
    r = jax.nn.sigmoid(gi[:, 0:_H] + gh[:, 0:_H])
    z = jax.nn.sigmoid(gi[:, _H:2 * _H] + gh[:, _H:2 * _H])
    n = jnp.tanh(gi[:, 2 * _H:3 * _H] + r * gh[:, 2 * _H:3 * _H])
    o_ref[0:_N, :] = (1.0 - z) * n + z * s
    o_ref[_N:_NP, :] = jnp.zeros((_NP - _N, _H), jnp.float32)


def _tc_gru(aggp, degp, s, root, cb, wihT, bih, whhT, bhh):
    return pl.pallas_call(
        _gru_body,
        out_shape=jax.ShapeDtypeStruct((_NP, _H), jnp.float32),
    )(aggp, degp, s, root, cb, wihT, bih, whhT, bhh)


def _s2s_body(s_ref, b_ref, wih_ref, bih_ref, whh_ref, bhh_ref,
              w1_ref, b1_ref, w2_ref, b2_ref, o_ref):
    s = s_ref[0:_N, :]
    bids = b_ref[...]
    iota = lax.broadcasted_iota(jnp.int32, (_N, _B), 1)
    msk = bids == iota
    mf = msk.astype(jnp.float32)
    q = jnp.zeros((_B, 2 * _H), jnp.float32)
    hh = jnp.zeros((_B, _H), jnp.float32)
    cc = jnp.zeros((_B, _H), jnp.float32)
    for _ in range(3):
        gates = (jnp.dot(q, wih_ref[...], preferred_element_type=jnp.float32)
                 + bih_ref[...]
                 + jnp.dot(hh, whh_ref[...], preferred_element_type=jnp.float32)
                 + bhh_ref[...])
        i_ = jax.nn.sigmoid(gates[:, 0:_H])
        f_ = jax.nn.sigmoid(gates[:, _H:2 * _H])
        g_ = jnp.tanh(gates[:, 2 * _H:3 * _H])
        o_ = jax.nn.sigmoid(gates[:, 3 * _H:4 * _H])
        cc = f_ * cc + i_ * g_
        hh = o_ * jnp.tanh(cc)
        hb = jnp.dot(mf, hh, preferred_element_type=jnp.float32)
        e = jnp.sum(s * hb, axis=1, keepdims=True)
        em = jnp.max(jnp.where(msk, e, -1e38), axis=0, keepdims=True)
        em = jnp.where(em < -1e37, 0.0, em)
        a = jnp.exp(e - jnp.sum(mf * em, axis=1, keepdims=True))
        asum = jnp.sum(mf * a, axis=0, keepdims=True)
        an = a / (jnp.sum(mf * asum, axis=1, keepdims=True) + 1e-16)
        r_ = lax.dot_general(mf * an, s, (((0,), (0,)), ((), ())),
                             preferred_element_type=jnp.float32)
        q = jnp.concatenate([hh, r_], axis=1)
    z1 = jax.nn.relu(
        jnp.dot(q, w1_ref[...], preferred_element_type=jnp.float32) + b1_ref[...])
    o_ref[...] = jnp.dot(z1, w2_ref[...], preferred_element_type=jnp.float32) + b2_ref[...]


def _tc_s2s(s, batch2d, lsWihT, lsbih, lsWhhT, lsbhh, w1, b1, w2, b2):
    return pl.pallas_call(
        _s2s_body,
        out_shape=jax.ShapeDtypeStruct((_B, 1), jnp.float32),
    )(s, batch2d, lsWihT, lsbih, lsWhhT, lsbhh, w1, b1, w2, b2)


def kernel(x, edge_index, batch, edge_attr, W0, b0, enW1, enb1, enW2, enb2,
           root, conv_bias, gru_Wih, gru_Whh, gru_bih, gru_bhh,
           ls_Wih, ls_Whh, ls_bih, ls_bhh, W1, b1, W2, b2):
    src = edge_index[0].astype(jnp.int32)
    dst = edge_index[1].astype(jnp.int32)
    src_p = jnp.concatenate([src, jnp.zeros((_EP - _E,), jnp.int32)])
    dst_p = jnp.concatenate([dst, jnp.full((_EP - _E,), _N, jnp.int32)])
    src_w = src_p.reshape(_NW, _CH, _CW)
    dst_w = dst_p.reshape(_NW, _CH, _CW)
    ea_p = jnp.pad(edge_attr, ((0, _EP - _E), (0, 0)))

    w2rb = (enW2.reshape(128, _H, _H).transpose(2, 1, 0)
            .reshape(_H, _H * 128).astype(jnp.bfloat16))
    b0m = enb2.reshape(_H, _H)
    z64 = jnp.zeros((_NP, _H), jnp.float32)
    z16 = jnp.zeros((_NP, 16), jnp.float32)
    ones = jnp.ones((_CW, 16), jnp.float32)

    s, ehb = _prep(x, W0, b0, ea_p, enW1, enb1)

    wihT = gru_Wih.T
    whhT = gru_Whh.T
    for _ in range(3):
        g = _sc_gather(s, src_w)
        msg = _tc_msg(ehb, g, w2rb, b0m)
        aggp, degp = _sc_scatter(msg, dst_w, z64, z16, ones)
        s = _tc_gru(aggp, degp, s, root, conv_bias, wihT, gru_bih,
                    whhT, gru_bhh)

    out = _tc_s2s(s, batch.astype(jnp.int32).reshape(_N, 1),
                  ls_Wih.T, ls_bih, ls_Whh.T, ls_bhh, W1, b1, W2, b2)
    return out.reshape(-1)


# speedup vs baseline: 3.9058x; 1.0625x over previous
import functools

import jax
import jax.numpy as jnp
from jax import lax
from jax.experimental import pallas as pl
from jax.experimental.pallas import tpu as pltpu
import jax.experimental.pallas.tpu_sc as plsc

_N, _E, _F, _H, _B = 5000, 30000, 128, 64, 256
_NC, _NS = 2, 16
_NW = _NC * _NS
_CW = 120
_CH = 8
_BPW = _CW * _CH
_EP = _BPW * _NW
_NP = 5120
_NPS = _NP // _NS
_MB = 1536


def _prep_body(x_ref, w0_ref, b0_ref, ea_ref, w1_ref, b1_ref, s0_ref, eh_ref):
    s0_ref[0:_N, :] = jax.nn.relu(
        jnp.dot(x_ref[...], w0_ref[...], preferred_element_type=jnp.float32)
        + b0_ref[...])
    s0_ref[_N:_NP, :] = jnp.zeros((_NP - _N, _H), jnp.float32)
    ehT = jax.nn.relu(
        lax.dot_general(w1_ref[...], ea_ref[...], (((0,), (1,)), ((), ())),
                        preferred_element_type=jnp.float32)
        + b1_ref[...][:, None])
    eh_ref[...] = ehT.astype(jnp.bfloat16)


def _prep(x, w0, b0, ea_p, enw1, enb1):
    return pl.pallas_call(
        _prep_body,
        out_shape=(jax.ShapeDtypeStruct((_NP, _H), jnp.float32),
                   jax.ShapeDtypeStruct((128, _EP), jnp.bfloat16)),
    )(x, w0, b0, ea_p, enw1, enb1)


def _gather_body(nodes_hbm, idx_hbm, g_hbm, idx_v, rows_v, stage_v, tab_sh, sem):
    c = lax.axis_index("c")
    s = lax.axis_index("s")
    wid = s * _NC + c
    r0 = s * _NPS
    pltpu.sync_copy(nodes_hbm.at[pl.ds(r0, _NPS)], stage_v)
    pltpu.sync_copy(stage_v, tab_sh.at[pl.ds(r0, _NPS)])
    plsc.subcore_barrier()
    pltpu.sync_copy(idx_hbm.at[wid], idx_v)
    cps = [
        pltpu.async_copy(tab_sh.at[idx_v.at[j]],
                         rows_v.at[pl.ds(j * _CW, _CW)], sem)
        for j in range(_CH)
    ]
    for cp in cps:
        cp.wait()
    pltpu.sync_copy(rows_v, g_hbm.at[pl.ds(wid * _BPW, _BPW)])


def _sc_gather(nodes, src_idx):
    mesh = plsc.VectorSubcoreMesh(core_axis_name="c", subcore_axis_name="s")
    return pl.kernel(
        _gather_body,
        out_type=jax.ShapeDtypeStruct((_EP, _H), jnp.float32),
        mesh=mesh,
        compiler_params=pltpu.CompilerParams(use_tc_tiling_on_sc=False),
        scratch_types=[
            pltpu.VMEM((_CH, _CW), jnp.int32),
            pltpu.VMEM((_BPW, _H), jnp.float32),
            pltpu.VMEM((_NPS, _H), jnp.float32),
            pltpu.VMEM_SHARED((_NP, _H), jnp.float32),
            pltpu.SemaphoreType.DMA,
        ],
    )(nodes, src_idx)


def _msg_body(eh_ref, g_ref, w_ref, b_ref, o_ref):
    g32 = g_ref[...]
    gT = g32.T.astype(jnp.bfloat16)
    ehT = eh_ref[...]
    p2 = (gT[:, None, :] * ehT[None, :, :]).reshape(_H * 128, _MB)
    msgT = jnp.dot(w_ref[...], p2, preferred_element_type=jnp.float32)
    acc = msgT.T + jnp.dot(g32, b_ref[...], preferred_element_type=jnp.float32)
    o_ref[...] = acc


def _tc_msg(ehb, g, w2rb, b0m):
    grid = _EP // _MB
    return pl.pallas_call(
        _msg_body,
        grid=(grid,),
        in_specs=[
            pl.BlockSpec((128, _MB), lambda i: (0, i)),
            pl.BlockSpec((_MB, _H), lambda i: (i, 0)),
            pl.BlockSpec((_H, _H * 128), lambda i: (0, 0)),
            pl.BlockSpec((_H, _H), lambda i: (0, 0)),
        ],
        out_specs=pl.BlockSpec((_MB, _H), lambda i: (i, 0)),
        out_shape=jax.ShapeDtypeStruct((_EP, _H), jnp.float32),
    )(ehb, g, w2rb, b0m)


def _scatter_body(msg_hbm, idx_hbm, z64_hbm, z16_hbm, ones_hbm,
                  agg_hbm, deg_hbm, idx_v, msg_v, ones_v, t16_v,
                  agg_sh, deg_sh):
    c = lax.axis_index("c")
    s = lax.axis_index("s")
    wid = s * _NC + c
    row0 = s * _NPS
    pltpu.sync_copy(z64_hbm.at[pl.ds(row0, _NPS)], msg_v.at[pl.ds(0, _NPS)])
    pltpu.sync_copy(msg_v.at[pl.ds(0, _NPS)], agg_sh.at[pl.ds(row0, _NPS)])
    pltpu.sync_copy(z16_hbm.at[pl.ds(row0, _NPS)], t16_v)
    pltpu.sync_copy(t16_v, deg_sh.at[pl.ds(row0, _NPS)])
    pltpu.sync_copy(ones_hbm, ones_v)
    plsc.subcore_barrier()
    pltpu.sync_copy(idx_hbm.at[wid], idx_v)
    pltpu.sync_copy(msg_hbm.at[pl.ds(wid * _BPW, _BPW)], msg_v)
    for j in range(_CH):
        pltpu.sync_copy(msg_v.at[pl.ds(j * _CW, _CW)],
                        agg_sh.at[idx_v.at[j]], add=True)
        pltpu.sync_copy(ones_v, deg_sh.at[idx_v.at[j]], add=True)
    plsc.subcore_barrier()
    pltpu.sync_copy(agg_sh.at[pl.ds(row0, _NPS)], msg_v.at[pl.ds(0, _NPS)])
    pltpu.sync_copy(msg_v.at[pl.ds(0, _NPS)],
                    agg_hbm.at[pl.ds(c * _NP + row0, _NPS)])
    pltpu.sync_copy(deg_sh.at[pl.ds(row0, _NPS)], t16_v)
    pltpu.sync_copy(t16_v, deg_hbm.at[pl.ds(c * _NP + row0, _NPS)])


def _sc_scatter(msg, dst_idx, z64, z16, ones):
    mesh = plsc.VectorSubcoreMesh(core_axis_name="c", subcore_axis_name="s")
    return pl.kernel(
        _scatter_body,
        out_type=(jax.ShapeDtypeStruct((_NC * _NP, _H), jnp.float32),
                  jax.ShapeDtypeStruct((_NC * _NP, 16), jnp.float32)),
        mesh=mesh,
        compiler_params=pltpu.CompilerParams(use_tc_tiling_on_sc=False),
        scratch_types=[
            pltpu.VMEM((_CH, _CW), jnp.int32),
            pltpu.VMEM((_BPW, _H), jnp.float32),
            pltpu.VMEM((_CW, 16), jnp.float32),
            pltpu.VMEM((_NPS, 16), jnp.float32),
            pltpu.VMEM_SHARED((_NP, _H), jnp.float32),
            pltpu.VMEM_SHARED((_NP, 16), jnp.float32),
        ],
    )(msg, dst_idx, z64, z16, ones)


def _gru_body(agg_ref, deg_ref, s_ref, root_ref, cb_ref, wih_ref, bih_ref,
              whh_ref, bhh_ref, o_ref):
    deg = jnp.clip(deg_ref[0:_N, 0:1] + deg_ref[_NP:_NP + _N, 0:1], 1.0, None)
    agg = (agg_ref[0:_N, :] + agg_ref[_NP:_NP + _N, :]) / deg
    s = s_ref[0:_N, :]
    m = jax.nn.relu(
        agg + jnp.dot(s, root_ref[...], preferred_element_type=jnp.float32)
        + cb_ref[...])
    gi = jnp.dot(m, wih_ref[...], preferred_element_type=jnp.float32) + bih_ref[...]
    gh = jnp.dot(s, whh_ref[...], preferred_element_type=jnp.float32) + bhh_ref[...]
    r = jax.nn.sigmoid(gi[:, 0:_H] + gh[:, 0:_H])
    z = jax.nn.sigmoid(gi[:, _H:2 * _H] + gh[:, _H:2 * _H])
    n = jnp.tanh(gi[:, 2 * _H:3 * _H] + r * gh[:, 2 * _H:3 * _H])
    o_ref[0:_N, :] = (1.0 - z) * n + z * s
    o_ref[_N:_NP, :] = jnp.zeros((_NP - _N, _H), jnp.float32)


def _tc_gru(aggp, degp, s, root, cb, wihT, bih, whhT, bhh):
    return pl.pallas_call(
        _gru_body,
        out_shape=jax.ShapeDtypeStruct((_NP, _H), jnp.float32),
    )(aggp, degp, s, root, cb, wihT, bih, whhT, bhh)


def _s2s_body(s_ref, b_ref, wih_ref, bih_ref, whh_ref, bhh_ref,
              w1_ref, b1_ref, w2_ref, b2_ref, o_ref):
    s = s_ref[0:_N, :]
    bids = b_ref[...]
    iota = lax.broadcasted_iota(jnp.int32, (_N, _B), 1)
    msk = bids == iota
    mf = msk.astype(jnp.float32)
    q = jnp.zeros((_B, 2 * _H), jnp.float32)
    hh = jnp.zeros((_B, _H), jnp.float32)
    cc = jnp.zeros((_B, _H), jnp.float32)
    for _ in range(3):
        gates = (jnp.dot(q, wih_ref[...], preferred_element_type=jnp.float32)
                 + bih_ref[...]
                 + jnp.dot(hh, whh_ref[...], preferred_element_type=jnp.float32)
                 + bhh_ref[...])
        i_ = jax.nn.sigmoid(gates[:, 0:_H])
        f_ = jax.nn.sigmoid(gates[:, _H:2 * _H])
        g_ = jnp.tanh(gates[:, 2 * _H:3 * _H])
        o_ = jax.nn.sigmoid(gates[:, 3 * _H:4 * _H])
        cc = f_ * cc + i_ * g_
        hh = o_ * jnp.tanh(cc)
        hb = jnp.dot(mf, hh, preferred_element_type=jnp.float32)
        e = jnp.sum(s * hb, axis=1, keepdims=True)
        em = jnp.max(jnp.where(msk, e, -1e38), axis=0, keepdims=True)
        em = jnp.where(em < -1e37, 0.0, em)
        a = jnp.exp(e - jnp.sum(mf * em, axis=1, keepdims=True))
        asum = jnp.sum(mf * a, axis=0, keepdims=True)
        an = a / (jnp.sum(mf * asum, axis=1, keepdims=True) + 1e-16)
        r_ = lax.dot_general(mf * an, s, (((0,), (0,)), ((), ())),
                             preferred_element_type=jnp.float32)
        q = jnp.concatenate([hh, r_], axis=1)
    z1 = jax.nn.relu(
        jnp.dot(q, w1_ref[...], preferred_element_type=jnp.float32) + b1_ref[...])
    o_ref[...] = jnp.dot(z1, w2_ref[...], preferred_element_type=jnp.float32) + b2_ref[...]


def _tc_s2s(s, batch2d, lsWihT, lsbih, lsWhhT, lsbhh, w1, b1, w2, b2):
    return pl.pallas_call(
        _s2s_body,
        out_shape=jax.ShapeDtypeStruct((_B, 1), jnp.float32),
    )(s, batch2d, lsWihT, lsbih, lsWhhT, lsbhh, w1, b1, w2, b2)


def kernel(x, edge_index, batch, edge_attr, W0, b0, enW1, enb1, enW2, enb2,
           root, conv_bias, gru_Wih, gru_Whh, gru_bih, gru_bhh,
           ls_Wih, ls_Whh, ls_bih, ls_bhh, W1, b1, W2, b2):
    src = edge_index[0].astype(jnp.int32)
    dst = edge_index[1].astype(jnp.int32)
    src_p = jnp.concatenate([src, jnp.zeros((_EP - _E,), jnp.int32)])
    dst_p = jnp.concatenate([dst, jnp.full((_EP - _E,), _N, jnp.int32)])
    src_w = src_p.reshape(_NW, _CH, _CW)
    dst_w = dst_p.reshape(_NW, _CH, _CW)
    ea_p = jnp.pad(edge_attr, ((0, _EP - _E), (0, 0)))

    w2rb = (enW2.reshape(128, _H, _H).transpose(2, 1, 0)
            .reshape(_H, _H * 128).astype(jnp.bfloat16))
    b0m = enb2.reshape(_H, _H)
    z64 = jnp.zeros((_NP, _H), jnp.float32)
    z16 = jnp.zeros((_NP, 16), jnp.float32)
    ones = jnp.ones((_CW, 16), jnp.float32)

    s, ehb = _prep(x, W0, b0, ea_p, enW1, enb1)

    wihT = gru_Wih.T
    whhT = gru_Whh.T
    for _ in range(3):
        g = _sc_gather(s, src_w)
        msg = _tc_msg(ehb, g, w2rb, b0m)
        aggp, degp = _sc_scatter(msg, dst_w, z64, z16, ones)
        s = _tc_gru(aggp, degp, s, root, conv_bias, wihT, gru_bih,
                    whhT, gru_bhh)

    out = _tc_s2s(s, batch.astype(jnp.int32).reshape(_N, 1),
                  ls_Wih.T, ls_bih, ls_Whh.T, ls_bhh, W1, b1, W2, b2)
    return out.reshape(-1)


# speedup vs baseline: 3.9901x; 1.0216x over previous
import functools

import jax
import jax.numpy as jnp
from jax import lax
from jax.experimental import pallas as pl
from jax.experimental.pallas import tpu as pltpu
import jax.experimental.pallas.tpu_sc as plsc

_N, _E, _F, _H, _B = 5000, 30000, 128, 64, 256
_NC, _NS = 2, 16
_NW = _NC * _NS
_CW = 120
_CH = 8
_BPW = _CW * _CH
_EP = _BPW * _NW
_NP = 5120
_NPS = _NP // _NS
_MB = 1536


def _prep_body(x_ref, w0_ref, b0_ref, ea_ref, w1_ref, b1_ref, s0_ref, eh_ref):
    s0_ref[0:_N, :] = jax.nn.relu(
        jnp.dot(x_ref[...], w0_ref[...], preferred_element_type=jnp.float32)
        + b0_ref[...])
    s0_ref[_N:_NP, :] = jnp.zeros((_NP - _N, _H), jnp.float32)
    ehT = jax.nn.relu(
        lax.dot_general(w1_ref[...], ea_ref[...], (((0,), (1,)), ((), ())),
                        preferred_element_type=jnp.float32)
        + b1_ref[...][:, None])
    eh_ref[:, 0:_E] = ehT.astype(jnp.bfloat16)
    eh_ref[:, _E:_EP] = jnp.zeros((128, _EP - _E), jnp.bfloat16)


def _prep(x, w0, b0, ea_p, enw1, enb1):
    return pl.pallas_call(
        _prep_body,
        out_shape=(jax.ShapeDtypeStruct((_NP, _H), jnp.float32),
                   jax.ShapeDtypeStruct((128, _EP), jnp.bfloat16)),
    )(x, w0, b0, ea_p, enw1, enb1)


def _gather_body(nodes_hbm, idx_hbm, g_hbm, idx_v, rows_v, stage_v, tab_sh, sem):
    c = lax.axis_index("c")
    s = lax.axis_index("s")
    wid = s * _NC + c
    r0 = s * _NPS
    pltpu.sync_copy(nodes_hbm.at[pl.ds(r0, _NPS)], stage_v)
    pltpu.sync_copy(stage_v, tab_sh.at[pl.ds(r0, _NPS)])
    plsc.subcore_barrier()
    pltpu.sync_copy(idx_hbm.at[wid], idx_v)
    cps = [
        pltpu.async_copy(tab_sh.at[idx_v.at[j]],
                         rows_v.at[pl.ds(j * _CW, _CW)], sem)
        for j in range(_CH)
    ]
    for cp in cps:
        cp.wait()
    pltpu.sync_copy(rows_v, g_hbm.at[pl.ds(wid * _BPW, _BPW)])


def _sc_gather(nodes, src_idx):
    mesh = plsc.VectorSubcoreMesh(core_axis_name="c", subcore_axis_name="s")
    return pl.kernel(
        _gather_body,
        out_type=jax.ShapeDtypeStruct((_EP, _H), jnp.float32),
        mesh=mesh,
        compiler_params=pltpu.CompilerParams(use_tc_tiling_on_sc=False),
        scratch_types=[
            pltpu.VMEM((_CH, _CW), jnp.int32),
            pltpu.VMEM((_BPW, _H), jnp.float32),
            pltpu.VMEM((_NPS, _H), jnp.float32),
            pltpu.VMEM_SHARED((_NP, _H), jnp.float32),
            pltpu.SemaphoreType.DMA,
        ],
    )(nodes, src_idx)


def _msg_body(eh_ref, g_ref, w_ref, b_ref, o_ref):
    g32 = g_ref[...]
    gT = g32.T.astype(jnp.bfloat16)
    ehT = eh_ref[...]
    p2 = (gT[:, None, :] * ehT[None, :, :]).reshape(_H * 128, _MB)
    msgT = jnp.dot(w_ref[...], p2, preferred_element_type=jnp.float32)
    acc = msgT.T + jnp.dot(g32, b_ref[...], preferred_element_type=jnp.float32)
    o_ref[...] = acc


def _tc_msg(ehb, g, w2rb, b0m):
    grid = _EP // _MB
    return pl.pallas_call(
        _msg_body,
        grid=(grid,),
        in_specs=[
            pl.BlockSpec((128, _MB), lambda i: (0, i)),
            pl.BlockSpec((_MB, _H), lambda i: (i, 0)),
            pl.BlockSpec((_H, _H * 128), lambda i: (0, 0)),
            pl.BlockSpec((_H, _H), lambda i: (0, 0)),
        ],
        out_specs=pl.BlockSpec((_MB, _H), lambda i: (i, 0)),
        out_shape=jax.ShapeDtypeStruct((_EP, _H), jnp.float32),
    )(ehb, g, w2rb, b0m)


def _scatter_deg_body(msg_hbm, idx_hbm, z64_hbm, z16_hbm, ones_hbm,
                      agg_hbm, deg_hbm, idx_v, msg_v, ones_v,
                      agg_sh, deg_sh):
    c = lax.axis_index("c")
    s = lax.axis_index("s")
    wid = s * _NC + c
    row0 = s * _NPS
    pltpu.sync_copy(z64_hbm.at[pl.ds(row0, _NPS)], agg_sh.at[pl.ds(row0, _NPS)])
    pltpu.sync_copy(z16_hbm.at[pl.ds(row0, _NPS)], deg_sh.at[pl.ds(row0, _NPS)])
    pltpu.sync_copy(ones_hbm, ones_v)
    plsc.subcore_barrier()
    pltpu.sync_copy(idx_hbm.at[wid], idx_v)
    pltpu.sync_copy(msg_hbm.at[pl.ds(wid * _BPW, _BPW)], msg_v)
    for j in range(_CH):
        pltpu.sync_copy(msg_v.at[pl.ds(j * _CW, _CW)],
                        agg_sh.at[idx_v.at[j]], add=True)
        pltpu.sync_copy(ones_v, deg_sh.at[idx_v.at[j]], add=True)
    plsc.subcore_barrier()
    pltpu.sync_copy(agg_sh.at[pl.ds(row0, _NPS)],
                    agg_hbm.at[pl.ds(c * _NP + row0, _NPS)])
    pltpu.sync_copy(deg_sh.at[pl.ds(row0, _NPS)],
                    deg_hbm.at[pl.ds(c * _NP + row0, _NPS)])


def _scatter_body(msg_hbm, idx_hbm, z64_hbm, agg_hbm, idx_v, msg_v, agg_sh):
    c = lax.axis_index("c")
    s = lax.axis_index("s")
    wid = s * _NC + c
    row0 = s * _NPS
    pltpu.sync_copy(z64_hbm.at[pl.ds(row0, _NPS)], agg_sh.at[pl.ds(row0, _NPS)])
    plsc.subcore_barrier()
    pltpu.sync_copy(idx_hbm.at[wid], idx_v)
    pltpu.sync_copy(msg_hbm.at[pl.ds(wid * _BPW, _BPW)], msg_v)
    for j in range(_CH):
        pltpu.sync_copy(msg_v.at[pl.ds(j * _CW, _CW)],
                        agg_sh.at[idx_v.at[j]], add=True)
    plsc.subcore_barrier()
    pltpu.sync_copy(agg_sh.at[pl.ds(row0, _NPS)],
                    agg_hbm.at[pl.ds(c * _NP + row0, _NPS)])


def _sc_scatter_deg(msg, dst_idx, z64, z16, ones):
    mesh = plsc.VectorSubcoreMesh(core_axis_name="c", subcore_axis_name="s")
    return pl.kernel(
        _scatter_deg_body,
        out_type=(jax.ShapeDtypeStruct((_NC * _NP, _H), jnp.float32),
                  jax.ShapeDtypeStruct((_NC * _NP, 16), jnp.float32)),
        mesh=mesh,
        compiler_params=pltpu.CompilerParams(use_tc_tiling_on_sc=False),
        scratch_types=[
            pltpu.VMEM((_CH, _CW), jnp.int32),
            pltpu.VMEM((_BPW, _H), jnp.float32),
            pltpu.VMEM((_CW, 16), jnp.float32),
            pltpu.VMEM_SHARED((_NP, _H), jnp.float32),
            pltpu.VMEM_SHARED((_NP, 16), jnp.float32),
        ],
    )(msg, dst_idx, z64, z16, ones)


def _sc_scatter(msg, dst_idx, z64):
    mesh = plsc.VectorSubcoreMesh(core_axis_name="c", subcore_axis_name="s")
    return pl.kernel(
        _scatter_body,
        out_type=jax.ShapeDtypeStruct((_NC * _NP, _H), jnp.float32),
        mesh=mesh,
        compiler_params=pltpu.CompilerParams(use_tc_tiling_on_sc=False),
        scratch_types=[
            pltpu.VMEM((_CH, _CW), jnp.int32),
            pltpu.VMEM((_BPW, _H), jnp.float32),
            pltpu.VMEM_SHARED((_NP, _H), jnp.float32),
        ],
    )(msg, dst_idx, z64)


def _gru_math(agg_ref, deg_ref, s, root_ref, cb_ref, wih_ref, bih_ref,
              whh_ref, bhh_ref):
    deg = jnp.clip(deg_ref[0:_N, 0:1] + deg_ref[_NP:_NP + _N, 0:1], 1.0, None)
    agg = (agg_ref[0:_N, :] + agg_ref[_NP:_NP + _N, :]) / deg
    m = jax.nn.relu(
        agg + jnp.dot(s, root_ref[...], preferred_element_type=jnp.float32)
        + cb_ref[...])
    gi = jnp.dot(m, wih_ref[...], preferred_element_type=jnp.float32) + bih_ref[...]
    gh = jnp.dot(s, whh_ref[...], preferred_element_type=jnp.float32) + bhh_ref[...]
    r = jax.nn.sigmoid(gi[:, 0:_H] + gh[:, 0:_H])
    z = jax.nn.sigmoid(gi[:, _H:2 * _H] + gh[:, _H:2 * _H])
    n = jnp.tanh(gi[:, 2 * _H:3 * _H] + r * gh[:, 2 * _H:3 * _H])
    return (1.0 - z) * n + z * s


def _gru_body(agg_ref, deg_ref, s_ref, root_ref, cb_ref, wih_ref, bih_ref,
              whh_ref, bhh_ref, o_ref):
    o_ref[0:_N, :] = _gru_math(agg_ref, deg_ref, s_ref[0:_N, :], root_ref,
                               cb_ref, wih_ref, bih_ref, whh_ref, bhh_ref)
    o_ref[_N:_NP, :] = jnp.zeros((_NP - _N, _H), jnp.float32)


def _tc_gru(aggp, degp, s, root, cb, wihT, bih, whhT, bhh):
    return pl.pallas_call(
        _gru_body,
        out_shape=jax.ShapeDtypeStruct((_NP, _H), jnp.float32),
    )(aggp, degp, s, root, cb, wihT, bih, whhT, bhh)


def _s2s_body(agg_ref, deg_ref, s_ref, root_ref, cb_ref, gwih_ref, gbih_ref,
              gwhh_ref, gbhh_ref, b_ref, wih_ref, bih_ref, whh_ref, bhh_ref,
              w1_ref, b1_ref, w2_ref, b2_ref, o_ref):
    s = _gru_math(agg_ref, deg_ref, s_ref[0:_N, :], root_ref, cb_ref,
                  gwih_ref, gbih_ref, gwhh_ref, gbhh_ref)
    bids = b_ref[...]
    iota = lax.broadcasted_iota(jnp.int32, (_N, _B), 1)
    msk = bids == iota
    mf = msk.astype(jnp.float32)
    q = jnp.zeros((_B, 2 * _H), jnp.float32)
    hh = jnp.zeros((_B, _H), jnp.float32)
    cc = jnp.zeros((_B, _H), jnp.float32)
    for _ in range(3):
        gates = (jnp.dot(q, wih_ref[...], preferred_element_type=jnp.float32)
                 + bih_ref[...]
                 + jnp.dot(hh, whh_ref[...], preferred_element_type=jnp.float32)
                 + bhh_ref[...])
        i_ = jax.nn.sigmoid(gates[:, 0:_H])
        f_ = jax.nn.sigmoid(gates[:, _H:2 * _H])
        g_ = jnp.tanh(gates[:, 2 * _H:3 * _H])
        o_ = jax.nn.sigmoid(gates[:, 3 * _H:4 * _H])
        cc = f_ * cc + i_ * g_
        hh = o_ * jnp.tanh(cc)
        hb = jnp.dot(mf, hh, preferred_element_type=jnp.float32)
        e = jnp.sum(s * hb, axis=1, keepdims=True)
        em = jnp.max(jnp.where(msk, e, -1e38), axis=0, keepdims=True)
        em = jnp.where(em < -1e37, 0.0, em)
        a = jnp.exp(e - jnp.sum(mf * em, axis=1, keepdims=True))
        asum = jnp.sum(mf * a, axis=0, keepdims=True)
        an = a / (jnp.sum(mf * asum, axis=1, keepdims=True) + 1e-16)
        r_ = lax.dot_general(mf * an, s, (((0,), (0,)), ((), ())),
                             preferred_element_type=jnp.float32)
        q = jnp.concatenate([hh, r_], axis=1)
    z1 = jax.nn.relu(
        jnp.dot(q, w1_ref[...], preferred_element_type=jnp.float32) + b1_ref[...])
    o_ref[...] = jnp.dot(z1, w2_ref[...], preferred_element_type=jnp.float32) + b2_ref[...]


def _tc_s2s(aggp, degp, s, root, cb, gwihT, gbih, gwhhT, gbhh,
            batch2d, lsWihT, lsbih, lsWhhT, lsbhh, w1, b1, w2, b2):
    return pl.pallas_call(
        _s2s_body,
        out_shape=jax.ShapeDtypeStruct((_B, 1), jnp.float32),
    )(aggp, degp, s, root, cb, gwihT, gbih, gwhhT, gbhh,
      batch2d, lsWihT, lsbih, lsWhhT, lsbhh, w1, b1, w2, b2)


def kernel(x, edge_index, batch, edge_attr, W0, b0, enW1, enb1, enW2, enb2,
           root, conv_bias, gru_Wih, gru_Whh, gru_bih, gru_bhh,
           ls_Wih, ls_Whh, ls_bih, ls_bhh, W1, b1, W2, b2):
    src = edge_index[0].astype(jnp.int32)
    dst = edge_index[1].astype(jnp.int32)
    src_p = jnp.concatenate([src, jnp.zeros((_EP - _E,), jnp.int32)])
    dst_p = jnp.concatenate([dst, jnp.full((_EP - _E,), _N, jnp.int32)])
    src_w = src_p.reshape(_NW, _CH, _CW)
    dst_w = dst_p.reshape(_NW, _CH, _CW)

    w2rb = (enW2.reshape(128, _H, _H).transpose(2, 1, 0)
            .reshape(_H, _H * 128).astype(jnp.bfloat16))
    b0m = enb2.reshape(_H, _H)
    z64 = jnp.zeros((_NP, _H), jnp.float32)
    z16 = jnp.zeros((_NP, 16), jnp.float32)
    ones = jnp.ones((_CW, 16), jnp.float32)

    s, ehb = _prep(x, W0, b0, edge_attr, enW1, enb1)

    wihT = gru_Wih.T
    whhT = gru_Whh.T
    degp = None
    for it in range(3):
        g = _sc_gather(s, src_w)
        msg = _tc_msg(ehb, g, w2rb, b0m)
        if it == 0:
            aggp, degp = _sc_scatter_deg(msg, dst_w, z64, z16, ones)
        else:
            aggp = _sc_scatter(msg, dst_w, z64)
        if it < 2:
            s = _tc_gru(aggp, degp, s, root, conv_bias, wihT, gru_bih,
                        whhT, gru_bhh)

    out = _tc_s2s(aggp, degp, s, root, conv_bias, wihT, gru_bih, whhT,
                  gru_bhh, batch.astype(jnp.int32).reshape(_N, 1),
                  ls_Wih.T, ls_bih, ls_Whh.T, ls_bhh, W1, b1, W2, b2)
    return out.reshape(-1)


# speedup vs baseline: 4.8086x; 1.2051x over previous
import functools

import jax
import jax.numpy as jnp
from jax import lax
from jax.experimental import pallas as pl
from jax.experimental.pallas import tpu as pltpu
import jax.experimental.pallas.tpu_sc as plsc

_N, _E, _F, _H, _B = 5000, 30000, 128, 64, 256
_NC, _NS = 2, 16
_NW = _NC * _NS
_CW = 120
_CH = 8
_BPW = _CW * _CH
_EP = _BPW * _NW
_NP = 5120
_NPS = _NP // _NS
_MB = 1536


def _prep_body(x_ref, w0_ref, b0_ref, ea_ref, w1_ref, b1_ref, s0_ref, eh_ref):
    s0_ref[0:_N, 0:_H] = jax.nn.relu(
        jnp.dot(x_ref[...], w0_ref[...], preferred_element_type=jnp.float32)
        + b0_ref[...])
    s0_ref[0:_N, _H:128] = jnp.zeros((_N, 128 - _H), jnp.float32)
    s0_ref[_N:_NP, :] = jnp.zeros((_NP - _N, 128), jnp.float32)
    ehT = jax.nn.relu(
        lax.dot_general(w1_ref[...], ea_ref[...], (((0,), (1,)), ((), ())),
                        preferred_element_type=jnp.float32)
        + b1_ref[...][:, None])
    eh_ref[:, 0:_E] = ehT.astype(jnp.bfloat16)
    eh_ref[:, _E:_EP] = jnp.zeros((128, _EP - _E), jnp.bfloat16)


def _prep(x, w0, b0, ea_p, enw1, enb1):
    return pl.pallas_call(
        _prep_body,
        out_shape=(jax.ShapeDtypeStruct((_NP, 128), jnp.float32),
                   jax.ShapeDtypeStruct((128, _EP), jnp.bfloat16)),
    )(x, w0, b0, ea_p, enw1, enb1)


def _gather_body(nodes_hbm, idx_hbm, g_hbm, idx_v, rows_v, tab_sh, sem):
    c = lax.axis_index("c")
    s = lax.axis_index("s")
    wid = s * _NC + c
    r0 = s * _NPS
    pltpu.sync_copy(nodes_hbm.at[pl.ds(r0, _NPS), 0:_H],
                    rows_v.at[pl.ds(0, _NPS)])
    pltpu.sync_copy(rows_v.at[pl.ds(0, _NPS)], tab_sh.at[pl.ds(r0, _NPS)])
    plsc.subcore_barrier()
    pltpu.sync_copy(idx_hbm.at[wid], idx_v)
    cps = [
        pltpu.async_copy(tab_sh.at[idx_v.at[j]],
                         rows_v.at[pl.ds(j * _CW, _CW)], sem)
        for j in range(_CH)
    ]
    for cp in cps:
        cp.wait()
    pltpu.sync_copy(rows_v, g_hbm.at[pl.ds(wid * _BPW, _BPW), 0:_H])


def _sc_gather(nodes, src_idx):
    mesh = plsc.VectorSubcoreMesh(core_axis_name="c", subcore_axis_name="s")
    return pl.kernel(
        _gather_body,
        out_type=jax.ShapeDtypeStruct((_EP, 128), jnp.float32),
        mesh=mesh,
        compiler_params=pltpu.CompilerParams(use_tc_tiling_on_sc=False),
        scratch_types=[
            pltpu.VMEM((_CH, _CW), jnp.int32),
            pltpu.VMEM((_BPW, _H), jnp.float32),
            pltpu.VMEM_SHARED((_NP, _H), jnp.float32),
            pltpu.SemaphoreType.DMA,
        ],
    )(nodes, src_idx)


def _msg_body(eh_ref, g_ref, w_ref, b_ref, o_ref):
    g32 = g_ref[:, 0:_H]
    gT = g32.T.astype(jnp.bfloat16)
    ehT = eh_ref[...]
    p2 = (gT[:, None, :] * ehT[None, :, :]).reshape(_H * 128, _MB)
    msgT = jnp.dot(w_ref[...], p2, preferred_element_type=jnp.float32)
    acc = msgT.T + jnp.dot(g32, b_ref[...], preferred_element_type=jnp.float32)
    o_ref[:, 0:_H] = acc
    o_ref[:, _H:128] = jnp.zeros((_MB, 128 - _H), jnp.float32)


def _tc_msg(ehb, g, w2rb, b0m):
    grid = _EP // _MB
    return pl.pallas_call(
        _msg_body,
        grid=(grid,),
        in_specs=[
            pl.BlockSpec((128, _MB), lambda i: (0, i)),
            pl.BlockSpec((_MB, 128), lambda i: (i, 0)),
            pl.BlockSpec((_H, _H * 128), lambda i: (0, 0)),
            pl.BlockSpec((_H, _H), lambda i: (0, 0)),
        ],
        out_specs=pl.BlockSpec((_MB, 128), lambda i: (i, 0)),
        out_shape=jax.ShapeDtypeStruct((_EP, 128), jnp.float32),
    )(ehb, g, w2rb, b0m)


def _scatter_body(msg_hbm, idx_hbm, z64_hbm, agg_hbm, idx_v, msg_v, agg_sh):
    c = lax.axis_index("c")
    s = lax.axis_index("s")
    wid = s * _NC + c
    row0 = s * _NPS
    pltpu.sync_copy(z64_hbm.at[pl.ds(0, _NPS)], agg_sh.at[pl.ds(row0, _NPS)])
    plsc.subcore_barrier()
    pltpu.sync_copy(idx_hbm.at[wid], idx_v)
    pltpu.sync_copy(msg_hbm.at[pl.ds(wid * _BPW, _BPW), 0:_H], msg_v)
    for j in range(_CH):
        pltpu.sync_copy(msg_v.at[pl.ds(j * _CW, _CW)],
                        agg_sh.at[idx_v.at[j]], add=True)
    plsc.subcore_barrier()
    pltpu.sync_copy(agg_sh.at[pl.ds(row0, _NPS)],
                    agg_hbm.at[pl.ds(c * _NP + row0, _NPS)])


def _deg_body(idx_hbm, z16_hbm, ones_hbm, deg_hbm, idx_v, ones_v, deg_sh):
    c = lax.axis_index("c")
    s = lax.axis_index("s")
    wid = s * _NC + c
    row0 = s * _NPS
    pltpu.sync_copy(z16_hbm, deg_sh.at[pl.ds(row0, _NPS)])
    pltpu.sync_copy(ones_hbm, ones_v)
    plsc.subcore_barrier()
    pltpu.sync_copy(idx_hbm.at[wid], idx_v)
    for j in range(_CH):
        pltpu.sync_copy(ones_v, deg_sh.at[idx_v.at[j]], add=True)
    plsc.subcore_barrier()
    pltpu.sync_copy(deg_sh.at[pl.ds(row0, _NPS)],
                    deg_hbm.at[pl.ds(c * _NP + row0, _NPS)])


def _sc_scatter(msg, dst_idx, z64):
    mesh = plsc.VectorSubcoreMesh(core_axis_name="c", subcore_axis_name="s")
    return pl.kernel(
        _scatter_body,
        out_type=jax.ShapeDtypeStruct((_NC * _NP, _H), jnp.float32),
        mesh=mesh,
        compiler_params=pltpu.CompilerParams(use_tc_tiling_on_sc=False),
        scratch_types=[
            pltpu.VMEM((_CH, _CW), jnp.int32),
            pltpu.VMEM((_BPW, _H), jnp.float32),
            pltpu.VMEM_SHARED((_NP, _H), jnp.float32),
        ],
    )(msg, dst_idx, z64)


def _sc_deg(dst_idx, z16, ones):
    mesh = plsc.VectorSubcoreMesh(core_axis_name="c", subcore_axis_name="s")
    return pl.kernel(
        _deg_body,
        out_type=jax.ShapeDtypeStruct((_NC * _NP, 16), jnp.float32),
        mesh=mesh,
        compiler_params=pltpu.CompilerParams(use_tc_tiling_on_sc=False),
        scratch_types=[
            pltpu.VMEM((_CH, _CW), jnp.int32),
            pltpu.VMEM((_CW, 16), jnp.float32),
            pltpu.VMEM_SHARED((_NP, 16), jnp.float32),
        ],
    )(dst_idx, z16, ones)


def _gru_math(agg_ref, deg_ref, s, root_ref, cb_ref, wih_ref, bih_ref,
              whh_ref, bhh_ref):
    deg = jnp.clip(deg_ref[0:_N, 0:1] + deg_ref[_NP:_NP + _N, 0:1], 1.0, None)
    agg = (agg_ref[0:_N, :] + agg_ref[_NP:_NP + _N, :]) / deg
    m = jax.nn.relu(
        agg + jnp.dot(s, root_ref[...], preferred_element_type=jnp.float32)
        + cb_ref[...])
    gi = jnp.dot(m, wih_ref[...], preferred_element_type=jnp.float32) + bih_ref[...]
    gh = jnp.dot(s, whh_ref[...], preferred_element_type=jnp.float32) + bhh_ref[...]
    r = jax.nn.sigmoid(gi[:, 0:_H] + gh[:, 0:_H])
    z = jax.nn.sigmoid(gi[:, _H:2 * _H] + gh[:, _H:2 * _H])
    n = jnp.tanh(gi[:, 2 * _H:3 * _H] + r * gh[:, 2 * _H:3 * _H])
    return (1.0 - z) * n + z * s


def _gru_body(agg_ref, deg_ref, s_ref, root_ref, cb_ref, wih_ref, bih_ref,
              whh_ref, bhh_ref, o_ref):
    o_ref[0:_N, 0:_H] = _gru_math(agg_ref, deg_ref, s_ref[0:_N, 0:_H],
                                  root_ref, cb_ref, wih_ref, bih_ref,
                                  whh_ref, bhh_ref)
    o_ref[0:_N, _H:128] = jnp.zeros((_N, 128 - _H), jnp.float32)
    o_ref[_N:_NP, :] = jnp.zeros((_NP - _N, 128), jnp.float32)


def _tc_gru(aggp, degp, s, root, cb, wihT, bih, whhT, bhh):
    return pl.pallas_call(
        _gru_body,
        out_shape=jax.ShapeDtypeStruct((_NP, 128), jnp.float32),
    )(aggp, degp, s, root, cb, wihT, bih, whhT, bhh)


def _s2s_body(agg_ref, deg_ref, s_ref, root_ref, cb_ref, gwih_ref, gbih_ref,
              gwhh_ref, gbhh_ref, b_ref, wih_ref, bih_ref, whh_ref, bhh_ref,
              w1_ref, b1_ref, w2_ref, b2_ref, o_ref):
    s = _gru_math(agg_ref, deg_ref, s_ref[0:_N, 0:_H], root_ref, cb_ref,
                  gwih_ref, gbih_ref, gwhh_ref, gbhh_ref)
    bids = b_ref[...]
    iota = lax.broadcasted_iota(jnp.int32, (_N, _B), 1)
    msk = bids == iota
    mf = msk.astype(jnp.float32)
    q = jnp.zeros((_B, 2 * _H), jnp.float32)
    hh = jnp.zeros((_B, _H), jnp.float32)
    cc = jnp.zeros((_B, _H), jnp.float32)
    for _ in range(3):
        gates = (jnp.dot(q, wih_ref[...], preferred_element_type=jnp.float32)
                 + bih_ref[...]
                 + jnp.dot(hh, whh_ref[...], preferred_element_type=jnp.float32)
                 + bhh_ref[...])
        i_ = jax.nn.sigmoid(gates[:, 0:_H])
        f_ = jax.nn.sigmoid(gates[:, _H:2 * _H])
        g_ = jnp.tanh(gates[:, 2 * _H:3 * _H])
        o_ = jax.nn.sigmoid(gates[:, 3 * _H:4 * _H])
        cc = f_ * cc + i_ * g_
        hh = o_ * jnp.tanh(cc)
        hb = jnp.dot(mf, hh, preferred_element_type=jnp.float32)
        e = jnp.sum(s * hb, axis=1, keepdims=True)
        em = jnp.max(jnp.where(msk, e, -1e38), axis=0, keepdims=True)
        em = jnp.where(em < -1e37, 0.0, em)
        a = jnp.exp(e - jnp.sum(mf * em, axis=1, keepdims=True))
        asum = jnp.sum(mf * a, axis=0, keepdims=True)
        an = a / (jnp.sum(mf * asum, axis=1, keepdims=True) + 1e-16)
        r_ = lax.dot_general(mf * an, s, (((0,), (0,)), ((), ())),
                             preferred_element_type=jnp.float32)
        q = jnp.concatenate([hh, r_], axis=1)
    z1 = jax.nn.relu(
        jnp.dot(q, w1_ref[...], preferred_element_type=jnp.float32) + b1_ref[...])
    o_ref[...] = jnp.dot(z1, w2_ref[...], preferred_element_type=jnp.float32) + b2_ref[...]


def _tc_s2s(aggp, degp, s, root, cb, gwihT, gbih, gwhhT, gbhh,
            batch2d, lsWihT, lsbih, lsWhhT, lsbhh, w1, b1, w2, b2):
    return pl.pallas_call(
        _s2s_body,
        out_shape=jax.ShapeDtypeStruct((_B, 1), jnp.float32),
    )(aggp, degp, s, root, cb, gwihT, gbih, gwhhT, gbhh,
      batch2d, lsWihT, lsbih, lsWhhT, lsbhh, w1, b1, w2, b2)


def kernel(x, edge_index, batch, edge_attr, W0, b0, enW1, enb1, enW2, enb2,
           root, conv_bias, gru_Wih, gru_Whh, gru_bih, gru_bhh,
           ls_Wih, ls_Whh, ls_bih, ls_bhh, W1, b1, W2, b2):
    src = edge_index[0].astype(jnp.int32)
    dst = edge_index[1].astype(jnp.int32)
    src_p = jnp.concatenate([src, jnp.zeros((_EP - _E,), jnp.int32)])
    dst_p = jnp.concatenate([dst, jnp.full((_EP - _E,), _N, jnp.int32)])
    src_w = src_p.reshape(_NW, _CH, _CW)
    dst_w = dst_p.reshape(_NW, _CH, _CW)

    w2rb = (enW2.reshape(128, _H, _H).transpose(2, 1, 0)
            .reshape(_H, _H * 128).astype(jnp.bfloat16))
    b0m = enb2.reshape(_H, _H)
    z64 = jnp.zeros((_NPS, _H), jnp.float32)
    z16 = jnp.zeros((_NPS, 16), jnp.float32)
    ones = jnp.ones((_CW, 16), jnp.float32)

    s, ehb = _prep(x, W0, b0, edge_attr, enW1, enb1)

    wihT = gru_Wih.T
    whhT = gru_Whh.T
    degp = _sc_deg(dst_w, z16, ones)
    for it in range(3):
        g = _sc_gather(s, src_w)
        msg = _tc_msg(ehb, g, w2rb, b0m)
        aggp = _sc_scatter(msg, dst_w, z64)
        if it < 2:
            s = _tc_gru(aggp, degp, s, root, conv_bias, wihT, gru_bih,
                        whhT, gru_bhh)

    out = _tc_s2s(aggp, degp, s, root, conv_bias, wihT, gru_bih, whhT,
                  gru_bhh, batch.astype(jnp.int32).reshape(_N, 1),
                  ls_Wih.T, ls_bih, ls_Whh.T, ls_bhh, W1, b1, W2, b2)
    return out.reshape(-1)


# speedup vs baseline: 4.8573x; 1.0101x over previous
import functools

import jax
import jax.numpy as jnp
from jax import lax
from jax.experimental import pallas as pl
from jax.experimental.pallas import tpu as pltpu
import jax.experimental.pallas.tpu_sc as plsc

_N, _E, _F, _H, _B = 5000, 30000, 128, 64, 256
_NC, _NS = 2, 16
_NW = _NC * _NS
_CW = 120
_CH = 8
_BPW = _CW * _CH
_EP = _BPW * _NW
_NP = 5120
_NPS = _NP // _NS
_MB = 1536


def _prep_body(x_ref, w0_ref, b0_ref, ea_ref, w1_ref, b1_ref, s0_ref, eh_ref):
    s0_ref[0:_N, 0:_H] = jax.nn.relu(
        jnp.dot(x_ref[...], w0_ref[...], preferred_element_type=jnp.float32)
        + b0_ref[...])
    s0_ref[0:_N, _H:128] = jnp.zeros((_N, 128 - _H), jnp.float32)
    s0_ref[_N:_NP, :] = jnp.zeros((_NP - _N, 128), jnp.float32)
    ehT = jax.nn.relu(
        lax.dot_general(w1_ref[...], ea_ref[...], (((0,), (1,)), ((), ())),
                        preferred_element_type=jnp.float32)
        + b1_ref[...][:, None])
    eh_ref[:, 0:_E] = ehT.astype(jnp.bfloat16)
    eh_ref[:, _E:_EP] = jnp.zeros((128, _EP - _E), jnp.bfloat16)


def _prep(x, w0, b0, ea_p, enw1, enb1):
    return pl.pallas_call(
        _prep_body,
        out_shape=(jax.ShapeDtypeStruct((_NP, 128), jnp.float32),
                   jax.ShapeDtypeStruct((128, _EP), jnp.bfloat16)),
    )(x, w0, b0, ea_p, enw1, enb1)


def _gather_body(nodes_hbm, idx_hbm, g_hbm, idx_v, rows_v, tab_sh, sem):
    c = lax.axis_index("c")
    s = lax.axis_index("s")
    wid = s * _NC + c
    r0 = s * _NPS
    pltpu.sync_copy(nodes_hbm.at[pl.ds(r0, _NPS), 0:_H],
                    rows_v.at[pl.ds(0, _NPS)])
    pltpu.sync_copy(rows_v.at[pl.ds(0, _NPS)], tab_sh.at[pl.ds(r0, _NPS)])
    plsc.subcore_barrier()
    pltpu.sync_copy(idx_hbm.at[wid], idx_v)
    cps = [
        pltpu.async_copy(tab_sh.at[idx_v.at[j]],
                         rows_v.at[pl.ds(j * _CW, _CW)], sem)
        for j in range(_CH)
    ]
    for cp in cps:
        cp.wait()
    pltpu.sync_copy(rows_v, g_hbm.at[pl.ds(wid * _BPW, _BPW), 0:_H])


def _sc_gather(nodes, src_idx):
    mesh = plsc.VectorSubcoreMesh(core_axis_name="c", subcore_axis_name="s")
    return pl.kernel(
        _gather_body,
        out_type=jax.ShapeDtypeStruct((_EP, 128), jnp.float32),
        mesh=mesh,
        compiler_params=pltpu.CompilerParams(use_tc_tiling_on_sc=False),
        scratch_types=[
            pltpu.VMEM((_CH, _CW), jnp.int32),
            pltpu.VMEM((_BPW, _H), jnp.float32),
            pltpu.VMEM_SHARED((_NP, _H), jnp.float32),
            pltpu.SemaphoreType.DMA,
        ],
    )(nodes, src_idx)


def _msg_body(eh_ref, g_ref, w_ref, b_ref, o_ref):
    g32 = g_ref[:, 0:_H]
    gT = g32.T.astype(jnp.bfloat16)
    ehT = eh_ref[...]
    p2 = (gT[:, None, :] * ehT[None, :, :]).reshape(_H * 128, _MB)
    msgT = jnp.dot(w_ref[...], p2, preferred_element_type=jnp.float32)
    acc = msgT.T + jnp.dot(g32, b_ref[...], preferred_element_type=jnp.float32)
    o_ref[:, 0:_H] = acc
    o_ref[:, _H:128] = jnp.zeros((_MB, 128 - _H), jnp.float32)


def _tc_msg(ehb, g, w2rb, b0m):
    grid = _EP // _MB
    return pl.pallas_call(
        _msg_body,
        grid=(grid,),
        in_specs=[
            pl.BlockSpec((128, _MB), lambda i: (0, i)),
            pl.BlockSpec((_MB, 128), lambda i: (i, 0)),
            pl.BlockSpec((_H, _H * 128), lambda i: (0, 0)),
            pl.BlockSpec((_H, _H), lambda i: (0, 0)),
        ],
        out_specs=pl.BlockSpec((_MB, 128), lambda i: (i, 0)),
        out_shape=jax.ShapeDtypeStruct((_EP, 128), jnp.float32),
    )(ehb, g, w2rb, b0m)


def _scatter_body(msg_hbm, idx_hbm, z64_hbm, agg_hbm, idx_v, msg_v, agg_sh,
                  sem):
    c = lax.axis_index("c")
    s = lax.axis_index("s")
    wid = s * _NC + c
    row0 = s * _NPS
    cp_i = pltpu.async_copy(idx_hbm.at[wid], idx_v, sem)
    cp_m = pltpu.async_copy(msg_hbm.at[pl.ds(wid * _BPW, _BPW), 0:_H],
                            msg_v, sem)
    pltpu.sync_copy(z64_hbm.at[pl.ds(0, _NPS)], agg_sh.at[pl.ds(row0, _NPS)])
    plsc.subcore_barrier()
    cp_i.wait()
    cp_m.wait()
    for j in range(_CH):
        pltpu.sync_copy(msg_v.at[pl.ds(j * _CW, _CW)],
                        agg_sh.at[idx_v.at[j]], add=True)
    plsc.subcore_barrier()
    pltpu.sync_copy(agg_sh.at[pl.ds(row0, _NPS)],
                    agg_hbm.at[pl.ds(c * _NP + row0, _NPS)])


def _deg_body(idx_hbm, z16_hbm, ones_hbm, deg_hbm, idx_v, ones_v, deg_sh):
    c = lax.axis_index("c")
    s = lax.axis_index("s")
    wid = s * _NC + c
    row0 = s * _NPS
    pltpu.sync_copy(z16_hbm, deg_sh.at[pl.ds(row0, _NPS)])
    pltpu.sync_copy(ones_hbm, ones_v)
    plsc.subcore_barrier()
    pltpu.sync_copy(idx_hbm.at[wid], idx_v)
    for j in range(_CH):
        pltpu.sync_copy(ones_v, deg_sh.at[idx_v.at[j]], add=True)
    plsc.subcore_barrier()
    pltpu.sync_copy(deg_sh.at[pl.ds(row0, _NPS)],
                    deg_hbm.at[pl.ds(c * _NP + row0, _NPS)])


def _sc_scatter(msg, dst_idx, z64):
    mesh = plsc.VectorSubcoreMesh(core_axis_name="c", subcore_axis_name="s")
    return pl.kernel(
        _scatter_body,
        out_type=jax.ShapeDtypeStruct((_NC * _NP, _H), jnp.float32),
        mesh=mesh,
        compiler_params=pltpu.CompilerParams(use_tc_tiling_on_sc=False),
        scratch_types=[
            pltpu.VMEM((_CH, _CW), jnp.int32),
            pltpu.VMEM((_BPW, _H), jnp.float32),
            pltpu.VMEM_SHARED((_NP, _H), jnp.float32),
            pltpu.SemaphoreType.DMA,
        ],
    )(msg, dst_idx, z64)


def _sc_deg(dst_idx, z16, ones):
    mesh = plsc.VectorSubcoreMesh(core_axis_name="c", subcore_axis_name="s")
    return pl.kernel(
        _deg_body,
        out_type=jax.ShapeDtypeStruct((_NC * _NP, 16), jnp.float32),
        mesh=mesh,
        compiler_params=pltpu.CompilerParams(use_tc_tiling_on_sc=False),
        scratch_types=[
            pltpu.VMEM((_CH, _CW), jnp.int32),
            pltpu.VMEM((_CW, 16), jnp.float32),
            pltpu.VMEM_SHARED((_NP, 16), jnp.float32),
        ],
    )(dst_idx, z16, ones)


def _gru_math(agg_ref, deg_ref, s, root_ref, cb_ref, wih_ref, bih_ref,
              whh_ref, bhh_ref):
    deg = jnp.clip(deg_ref[0:_N, 0:1] + deg_ref[_NP:_NP + _N, 0:1], 1.0, None)
    agg = (agg_ref[0:_N, :] + agg_ref[_NP:_NP + _N, :]) / deg
    m = jax.nn.relu(
        agg + jnp.dot(s, root_ref[...], preferred_element_type=jnp.float32)
        + cb_ref[...])
    gi = jnp.dot(m, wih_ref[...], preferred_element_type=jnp.float32) + bih_ref[...]
    gh = jnp.dot(s, whh_ref[...], preferred_element_type=jnp.float32) + bhh_ref[...]
    r = jax.nn.sigmoid(gi[:, 0:_H] + gh[:, 0:_H])
    z = jax.nn.sigmoid(gi[:, _H:2 * _H] + gh[:, _H:2 * _H])
    n = jnp.tanh(gi[:, 2 * _H:3 * _H] + r * gh[:, 2 * _H:3 * _H])
    return (1.0 - z) * n + z * s


def _gru_body(agg_ref, deg_ref, s_ref, root_ref, cb_ref, wih_ref, bih_ref,
              whh_ref, bhh_ref, o_ref):
    o_ref[0:_N, 0:_H] = _gru_math(agg_ref, deg_ref, s_ref[0:_N, 0:_H],
                                  root_ref, cb_ref, wih_ref, bih_ref,
                                  whh_ref, bhh_ref)
    o_ref[0:_N, _H:128] = jnp.zeros((_N, 128 - _H), jnp.float32)
    o_ref[_N:_NP, :] = jnp.zeros((_NP - _N, 128), jnp.float32)


def _tc_gru(aggp, degp, s, root, cb, wihT, bih, whhT, bhh):
    return pl.pallas_call(
        _gru_body,
        out_shape=jax.ShapeDtypeStruct((_NP, 128), jnp.float32),
    )(aggp, degp, s, root, cb, wihT, bih, whhT, bhh)


def _s2s_body(agg_ref, deg_ref, s_ref, root_ref, cb_ref, gwih_ref, gbih_ref,
              gwhh_ref, gbhh_ref, b_ref, wih_ref, bih_ref, whh_ref, bhh_ref,
              w1_ref, b1_ref, w2_ref, b2_ref, o_ref):
    s = _gru_math(agg_ref, deg_ref, s_ref[0:_N, 0:_H], root_ref, cb_ref,
                  gwih_ref, gbih_ref, gwhh_ref, gbhh_ref)
    bids = b_ref[...]
    iota = lax.broadcasted_iota(jnp.int32, (_N, _B), 1)
    msk = bids == iota
    mf = msk.astype(jnp.float32)
    q = jnp.zeros((_B, 2 * _H), jnp.float32)
    hh = jnp.zeros((_B, _H), jnp.float32)
    cc = jnp.zeros((_B, _H), jnp.float32)
    for _ in range(3):
        gates = (jnp.dot(q, wih_ref[...], preferred_element_type=jnp.float32)
                 + bih_ref[...]
                 + jnp.dot(hh, whh_ref[...], preferred_element_type=jnp.float32)
                 + bhh_ref[...])
        i_ = jax.nn.sigmoid(gates[:, 0:_H])
        f_ = jax.nn.sigmoid(gates[:, _H:2 * _H])
        g_ = jnp.tanh(gates[:, 2 * _H:3 * _H])
        o_ = jax.nn.sigmoid(gates[:, 3 * _H:4 * _H])
        cc = f_ * cc + i_ * g_
        hh = o_ * jnp.tanh(cc)
        hb = jnp.dot(mf, hh, preferred_element_type=jnp.float32)
        e = jnp.sum(s * hb, axis=1, keepdims=True)
        em = jnp.max(jnp.where(msk, e, -1e38), axis=0, keepdims=True)
        em = jnp.where(em < -1e37, 0.0, em)
        a = jnp.exp(e - jnp.sum(mf * em, axis=1, keepdims=True))
        asum = jnp.sum(mf * a, axis=0, keepdims=True)
        an = a / (jnp.sum(mf * asum, axis=1, keepdims=True) + 1e-16)
        r_ = lax.dot_general(mf * an, s, (((0,), (0,)), ((), ())),
                             preferred_element_type=jnp.float32)
        q = jnp.concatenate([hh, r_], axis=1)
    z1 = jax.nn.relu(
        jnp.dot(q, w1_ref[...], preferred_element_type=jnp.float32) + b1_ref[...])
    o_ref[...] = jnp.dot(z1, w2_ref[...], preferred_element_type=jnp.float32) + b2_ref[...]


def _tc_s2s(aggp, degp, s, root, cb, gwihT, gbih, gwhhT, gbhh,
            batch2d, lsWihT, lsbih, lsWhhT, lsbhh, w1, b1, w2, b2):
    return pl.pallas_call(
        _s2s_body,
        out_shape=jax.ShapeDtypeStruct((_B, 1), jnp.float32),
    )(aggp, degp, s, root, cb, gwihT, gbih, gwhhT, gbhh,
      batch2d, lsWihT, lsbih, lsWhhT, lsbhh, w1, b1, w2, b2)


def kernel(x, edge_index, batch, edge_attr, W0, b0, enW1, enb1, enW2, enb2,
           root, conv_bias, gru_Wih, gru_Whh, gru_bih, gru_bhh,
           ls_Wih, ls_Whh, ls_bih, ls_bhh, W1, b1, W2, b2):
    src = edge_index[0].astype(jnp.int32)
    dst = edge_index[1].astype(jnp.int32)
    src_p = jnp.concatenate([src, jnp.zeros((_EP - _E,), jnp.int32)])
    dst_p = jnp.concatenate([dst, jnp.full((_EP - _E,), _N, jnp.int32)])
    src_w = src_p.reshape(_NW, _CH, _CW)
    dst_w = dst_p.reshape(_NW, _CH, _CW)

    w2rb = (enW2.reshape(128, _H, _H).transpose(2, 1, 0)
            .reshape(_H, _H * 128).astype(jnp.bfloat16))
    b0m = enb2.reshape(_H, _H)
    z64 = jnp.zeros((_NPS, _H), jnp.float32)
    z16 = jnp.zeros((_NPS, 16), jnp.float32)
    ones = jnp.ones((_CW, 16), jnp.float32)

    s, ehb = _prep(x, W0, b0, edge_attr, enW1, enb1)

    wihT = gru_Wih.T
    whhT = gru_Whh.T
    degp = _sc_deg(dst_w, z16, ones)
    for it in range(3):
        g = _sc_gather(s, src_w)
        msg = _tc_msg(ehb, g, w2rb, b0m)
        aggp = _sc_scatter(msg, dst_w, z64)
        if it < 2:
            s = _tc_gru(aggp, degp, s, root, conv_bias, wihT, gru_bih,
                        whhT, gru_bhh)

    out = _tc_s2s(aggp, degp, s, root, conv_bias, wihT, gru_bih, whhT,
                  gru_bhh, batch.astype(jnp.int32).reshape(_N, 1),
                  ls_Wih.T, ls_bih, ls_Whh.T, ls_bhh, W1, b1, W2, b2)
    return out.reshape(-1)
